# baseline (speedup 1.0000x reference)
import jax
import jax.numpy as jnp
from jax import lax
from jax.experimental import pallas as pl
from jax.experimental.pallas import tpu as pltpu
from jax.experimental.pallas import tpu_sc as plsc

N_NODES = 100000
D = 128
N_POOL = 200000
NC, NS = 2, 16
NW = NC * NS

NCW = 8
XC = 200
XCPW = 62
XSPAN = XC * XCPW
XTAIL = (N_NODES - NCW * XSPAN) // XC

NPW = NW - NCW
PC = 64
GC = 2 * PC
CPW = 130
WSPAN = CPW * PC
FSPAN = 2 * WSPAN
NTAIL = (N_POOL - NPW * WSPAN) // PC


def _sc_body(x_hbm, iv_hbm, out_hbm,
             iv, g_v, o_v, xbuf, gsem0, gsem1, ssem0, ssem1):
    w = lax.axis_index("s") * NC + lax.axis_index("c")
    gsem = [gsem0, gsem1]
    ssem = [ssem0, ssem1]

    @pl.when(w < NCW)
    def _():
        base = w * XSPAN
        xb = [xbuf.at[0], xbuf.at[1]]

        def fire(k, p):
            pltpu.async_copy(x_hbm.at[pl.ds(base + k * XC, XC), :], xb[p],
                             gsem[p])

        def consume(k, p):
            pltpu.make_async_copy(x_hbm.at[pl.ds(base + k * XC, XC), :],
                                  xb[p], gsem[p]).wait()
            pltpu.async_copy(xb[p], out_hbm.at[pl.ds(base + k * XC, XC), :],
                             ssem[p])

        def wait_store(k, p):
            pltpu.make_async_copy(xb[p],
                                  out_hbm.at[pl.ds(base + k * XC, XC), :],
                                  ssem[p]).wait()

        fire(0, 0)

        def pipe(t, carry):
            k1 = 2 * t + 1

            @pl.when(t >= 1)
            def _():
                wait_store(k1 - 2, 1)

            fire(k1, 1)
            consume(k1 - 1, 0)

            k2 = 2 * t + 2
            wait_store(k2 - 2, 0)
            fire(k2, 0)
            consume(k2 - 1, 1)
            return carry

        lax.fori_loop(0, (XCPW - 2) // 2, pipe, 0)
        wait_store(XCPW - 3, 1)
        fire(XCPW - 1, 1)
        consume(XCPW - 2, 0)
        consume(XCPW - 1, 1)
        wait_store(XCPW - 2, 0)
        wait_store(XCPW - 1, 1)

        @pl.when(w < XTAIL)
        def _():
            tb = NCW * XSPAN + w * XC
            pltpu.sync_copy(x_hbm.at[pl.ds(tb, XC), :], xb[0])
            pltpu.sync_copy(xb[0], out_hbm.at[pl.ds(tb, XC), :])

    @pl.when(w >= NCW)
    def _():
        wp = w - NCW
        base = wp * WSPAN
        fbase = wp * FSPAN
        pltpu.sync_copy(iv_hbm.at[pl.ds(fbase, FSPAN)], iv)

        gv = [g_v.at[0], g_v.at[1]]
        ov = [o_v.at[0], o_v.at[1]]

        def fire(k, p):
            pltpu.async_copy(x_hbm.at[iv.at[pl.ds(k * GC, GC)]], gv[p],
                             gsem[p])

        def compute(p):
            def row(i, carry):
                for j in range(D // 16):
                    s = pl.ds(j * 16, 16)
                    o_v[p, i, s] = (g_v[p, 2 * i, s] + g_v[p, 2 * i + 1, s]) * 0.5
                return carry

            lax.fori_loop(0, PC, row, 0)

        def consume(k, p):
            pltpu.make_async_copy(x_hbm.at[iv.at[pl.ds(k * GC, GC)]], gv[p],
                                  gsem[p]).wait()
            compute(p)
            pltpu.async_copy(ov[p],
                             out_hbm.at[pl.ds(N_NODES + base + k * PC, PC), :],
                             ssem[p])

        def wait_store(k, p):
            pltpu.make_async_copy(ov[p],
                                  out_hbm.at[pl.ds(N_NODES + base + k * PC, PC), :],
                                  ssem[p]).wait()

        fire(0, 0)

        def pipe(t, carry):
            k1 = 2 * t + 1

            @pl.when(t >= 1)
            def _():
                wait_store(k1 - 2, 1)

            fire(k1, 1)
            consume(k1 - 1, 0)

            k2 = 2 * t + 2
            wait_store(k2 - 2, 0)
            fire(k2, 0)
            consume(k2 - 1, 1)
            return carry

        lax.fori_loop(0, (CPW - 2) // 2, pipe, 0)
        wait_store(CPW - 3, 1)
        fire(CPW - 1, 1)
        consume(CPW - 2, 0)
        consume(CPW - 1, 1)
        wait_store(CPW - 2, 0)
        wait_store(CPW - 1, 1)

        @pl.when(wp < NTAIL)
        def _():
            tbase = NPW * WSPAN + wp * PC
            tf = NPW * FSPAN + wp * GC
            pltpu.sync_copy(iv_hbm.at[pl.ds(tf, GC)], iv.at[pl.ds(0, GC)])
            pltpu.async_copy(x_hbm.at[iv.at[pl.ds(0, GC)]], gv[0], gsem[0])
            pltpu.make_async_copy(x_hbm.at[iv.at[pl.ds(0, GC)]], gv[0],
                                  gsem[0]).wait()
            compute(0)
            pltpu.sync_copy(ov[0], out_hbm.at[pl.ds(N_NODES + tbase, PC), :])


def kernel(X, pool_idx):
    iv = pool_idx.reshape(-1)
    mesh = plsc.VectorSubcoreMesh(core_axis_name="c", subcore_axis_name="s")
    f = pl.kernel(
        _sc_body,
        out_type=jax.ShapeDtypeStruct((N_NODES + N_POOL, D), jnp.float32),
        mesh=mesh,
        scratch_types=[
            pltpu.VMEM((FSPAN,), jnp.int32),
            pltpu.VMEM((2, GC, D), jnp.float32),
            pltpu.VMEM((2, PC, D), jnp.float32),
            pltpu.VMEM((2, XC, D), jnp.float32),
            pltpu.SemaphoreType.DMA,
            pltpu.SemaphoreType.DMA,
            pltpu.SemaphoreType.DMA,
            pltpu.SemaphoreType.DMA,
        ],
    )
    return f(X, iv)


# speedup vs baseline: 3.2203x; 3.2203x over previous
import jax
import jax.numpy as jnp
from jax import lax
from jax.experimental import pallas as pl
from jax.experimental.pallas import tpu as pltpu
from jax.experimental.pallas import tpu_sc as plsc

N_NODES = 100000
D = 128
N_POOL = 200000
NC, NS = 2, 16
NW = NC * NS

NCW = 8
XC = 200
XCPW = 62
XSPAN = XC * XCPW
XTAIL = (N_NODES - NCW * XSPAN) // XC

NPW = NW - NCW
PC = 104
PH1 = 56
PH2 = PC - PH1
CPW = 80
WSPAN = CPW * PC
PT = 80
NTAIL = (N_POOL - NPW * WSPAN) // PT


def _sc_body(x_hbm, i0_hbm, i1_hbm, out_hbm,
             i0v, i1v, a_v, b_v, xbuf, gsem0, gsem1, ssem0, ssem1):
    w = lax.axis_index("s") * NC + lax.axis_index("c")
    gsem = [gsem0, gsem1]
    ssem = [ssem0, ssem1]

    @pl.when(w < NCW)
    def _():
        base = w * XSPAN
        xb = [xbuf.at[0], xbuf.at[1]]

        def fire(k, p):
            pltpu.async_copy(x_hbm.at[pl.ds(base + k * XC, XC), :], xb[p],
                             gsem[p])

        def consume(k, p):
            pltpu.make_async_copy(x_hbm.at[pl.ds(base + k * XC, XC), :],
                                  xb[p], gsem[p]).wait()
            pltpu.async_copy(xb[p], out_hbm.at[pl.ds(base + k * XC, XC), :],
                             ssem[p])

        def wait_store(k, p):
            pltpu.make_async_copy(xb[p],
                                  out_hbm.at[pl.ds(base + k * XC, XC), :],
                                  ssem[p]).wait()

        fire(0, 0)

        def pipe(t, carry):
            k1 = 2 * t + 1

            @pl.when(t >= 1)
            def _():
                wait_store(k1 - 2, 1)

            fire(k1, 1)
            consume(k1 - 1, 0)

            k2 = 2 * t + 2
            wait_store(k2 - 2, 0)
            fire(k2, 0)
            consume(k2 - 1, 1)
            return carry

        lax.fori_loop(0, (XCPW - 2) // 2, pipe, 0)
        wait_store(XCPW - 3, 1)
        fire(XCPW - 1, 1)
        consume(XCPW - 2, 0)
        consume(XCPW - 1, 1)
        wait_store(XCPW - 2, 0)
        wait_store(XCPW - 1, 1)

        @pl.when(w < XTAIL)
        def _():
            tb = NCW * XSPAN + w * XC
            pltpu.sync_copy(x_hbm.at[pl.ds(tb, XC), :], xb[0])
            pltpu.sync_copy(xb[0], out_hbm.at[pl.ds(tb, XC), :])

    @pl.when(w >= NCW)
    def _():
        wp = w - NCW
        base = wp * WSPAN
        pltpu.sync_copy(i0_hbm.at[pl.ds(base, WSPAN)], i0v)
        pltpu.sync_copy(i1_hbm.at[pl.ds(base, WSPAN)], i1v)

        av = [a_v.at[0], a_v.at[1]]
        bv = [b_v.at[0], b_v.at[1]]

        def gather_descs(k, p):
            off = k * PC
            return [
                pltpu.make_async_copy(x_hbm.at[i0v.at[pl.ds(off, PH1)]],
                                      a_v.at[p, pl.ds(0, PH1), :], gsem[p]),
                pltpu.make_async_copy(x_hbm.at[i0v.at[pl.ds(off + PH1, PH2)]],
                                      a_v.at[p, pl.ds(PH1, PH2), :], gsem[p]),
                pltpu.make_async_copy(x_hbm.at[i1v.at[pl.ds(off, PH1)]],
                                      b_v.at[p, pl.ds(0, PH1), :], gsem[p]),
                pltpu.make_async_copy(x_hbm.at[i1v.at[pl.ds(off + PH1, PH2)]],
                                      b_v.at[p, pl.ds(PH1, PH2), :], gsem[p]),
            ]

        def fire(k, p):
            for d in gather_descs(k, p):
                d.start()

        def compute(p):
            def row(i, carry):
                for j in range(D // 16):
                    s = pl.ds(j * 16, 16)
                    a_v[p, i, s] = (a_v[p, i, s] + b_v[p, i, s]) * 0.5
                return carry

            lax.fori_loop(0, PC, row, 0)

        def consume(k, p):
            for d in gather_descs(k, p):
                d.wait()
            compute(p)
            pltpu.async_copy(av[p],
                             out_hbm.at[pl.ds(N_NODES + base + k * PC, PC), :],
                             ssem[p])

        def wait_store(k, p):
            pltpu.make_async_copy(av[p],
                                  out_hbm.at[pl.ds(N_NODES + base + k * PC, PC), :],
                                  ssem[p]).wait()

        fire(0, 0)

        def pipe(t, carry):
            k1 = 2 * t + 1

            @pl.when(t >= 1)
            def _():
                wait_store(k1 - 2, 1)

            fire(k1, 1)
            consume(k1 - 1, 0)

            k2 = 2 * t + 2
            wait_store(k2 - 2, 0)
            fire(k2, 0)
            consume(k2 - 1, 1)
            return carry

        lax.fori_loop(0, (CPW - 2) // 2, pipe, 0)
        wait_store(CPW - 3, 1)
        fire(CPW - 1, 1)
        consume(CPW - 2, 0)
        consume(CPW - 1, 1)
        wait_store(CPW - 2, 0)
        wait_store(CPW - 1, 1)

        @pl.when(wp < NTAIL)
        def _():
            tbase = NPW * WSPAN + wp * PT
            av0 = a_v.at[0, pl.ds(0, PT), :]
            bv0 = b_v.at[0, pl.ds(0, PT), :]
            pltpu.sync_copy(i0_hbm.at[pl.ds(tbase, PT)], i0v.at[pl.ds(0, PT)])
            pltpu.sync_copy(i1_hbm.at[pl.ds(tbase, PT)], i1v.at[pl.ds(0, PT)])
            pltpu.async_copy(x_hbm.at[i0v.at[pl.ds(0, PT)]], av0, gsem[0])
            pltpu.async_copy(x_hbm.at[i1v.at[pl.ds(0, PT)]], bv0, gsem[0])
            pltpu.make_async_copy(x_hbm.at[i0v.at[pl.ds(0, PT)]], av0,
                                  gsem[0]).wait()
            pltpu.make_async_copy(x_hbm.at[i1v.at[pl.ds(0, PT)]], bv0,
                                  gsem[0]).wait()

            def trow(i, carry):
                for j in range(D // 16):
                    s = pl.ds(j * 16, 16)
                    a_v[0, i, s] = (a_v[0, i, s] + b_v[0, i, s]) * 0.5
                return carry

            lax.fori_loop(0, PT, trow, 0)
            pltpu.sync_copy(av0, out_hbm.at[pl.ds(N_NODES + tbase, PT), :])


def kernel(X, pool_idx):
    idx0 = pool_idx[:, 0]
    idx1 = pool_idx[:, 1]
    mesh = plsc.VectorSubcoreMesh(core_axis_name="c", subcore_axis_name="s")
    f = pl.kernel(
        _sc_body,
        out_type=jax.ShapeDtypeStruct((N_NODES + N_POOL, D), jnp.float32),
        mesh=mesh,
        scratch_types=[
            pltpu.VMEM((WSPAN,), jnp.int32),
            pltpu.VMEM((WSPAN,), jnp.int32),
            pltpu.VMEM((2, PC, D), jnp.float32),
            pltpu.VMEM((2, PC, D), jnp.float32),
            pltpu.VMEM((2, XC, D), jnp.float32),
            pltpu.SemaphoreType.DMA,
            pltpu.SemaphoreType.DMA,
            pltpu.SemaphoreType.DMA,
            pltpu.SemaphoreType.DMA,
        ],
    )
    return f(X, idx0, idx1)


# speedup vs baseline: 3.3279x; 1.0334x over previous
import jax
import jax.numpy as jnp
from jax import lax
from jax.experimental import pallas as pl
from jax.experimental.pallas import tpu as pltpu
from jax.experimental.pallas import tpu_sc as plsc

N_NODES = 100000
D = 128
N_POOL = 200000
NC, NS = 2, 16
NW = NC * NS

NCW = 8
XC = 160
XCPW = 78
XSPAN = XC * XCPW
XTAIL = (N_NODES - NCW * XSPAN) // XC

NPW = NW - NCW
PC = 128
CPW = 65
WSPAN = CPW * PC
PT = 80
NTAIL = (N_POOL - NPW * WSPAN) // PT


def _sc_body(x_hbm, i0_hbm, i1_hbm, out_hbm,
             i0v, i1v, a_v, b_v, xbuf, gsem0, gsem1, ssem0, ssem1):
    w = lax.axis_index("s") * NC + lax.axis_index("c")
    gsem = [gsem0, gsem1]
    ssem = [ssem0, ssem1]

    @pl.when(w < NCW)
    def _():
        base = w * XSPAN
        xb = [xbuf.at[0], xbuf.at[1]]

        def fire(k, p):
            pltpu.async_copy(x_hbm.at[pl.ds(base + k * XC, XC), :], xb[p],
                             gsem[p])

        def consume(k, p):
            pltpu.make_async_copy(x_hbm.at[pl.ds(base + k * XC, XC), :],
                                  xb[p], gsem[p]).wait()
            pltpu.async_copy(xb[p], out_hbm.at[pl.ds(base + k * XC, XC), :],
                             ssem[p])

        def wait_store(k, p):
            pltpu.make_async_copy(xb[p],
                                  out_hbm.at[pl.ds(base + k * XC, XC), :],
                                  ssem[p]).wait()

        fire(0, 0)

        def pipe(t, carry):
            k1 = 2 * t + 1

            @pl.when(t >= 1)
            def _():
                wait_store(k1 - 2, 1)

            fire(k1, 1)
            consume(k1 - 1, 0)

            k2 = 2 * t + 2
            wait_store(k2 - 2, 0)
            fire(k2, 0)
            consume(k2 - 1, 1)
            return carry

        lax.fori_loop(0, (XCPW - 2) // 2, pipe, 0)
        wait_store(XCPW - 3, 1)
        fire(XCPW - 1, 1)
        consume(XCPW - 2, 0)
        consume(XCPW - 1, 1)
        wait_store(XCPW - 2, 0)
        wait_store(XCPW - 1, 1)

        @pl.when(w < XTAIL)
        def _():
            tb = NCW * XSPAN + w * XC
            pltpu.sync_copy(x_hbm.at[pl.ds(tb, XC), :], xb[0])
            pltpu.sync_copy(xb[0], out_hbm.at[pl.ds(tb, XC), :])

    @pl.when(w >= NCW)
    def _():
        wp = w - NCW
        base = wp * WSPAN
        pltpu.sync_copy(i0_hbm.at[pl.ds(base, WSPAN)], i0v)
        pltpu.sync_copy(i1_hbm.at[pl.ds(base, WSPAN)], i1v)

        av = [a_v.at[0], a_v.at[1]]
        bv = [b_v.at[0], b_v.at[1]]

        def gather_descs(k, p):
            off = k * PC
            return [
                pltpu.make_async_copy(x_hbm.at[i0v.at[pl.ds(off, PC)]],
                                      av[p], gsem[p]),
                pltpu.make_async_copy(x_hbm.at[i1v.at[pl.ds(off, PC)]],
                                      bv[p], gsem[p]),
            ]

        def fire(k, p):
            for d in gather_descs(k, p):
                d.start()

        def compute(p):
            def row(i, carry):
                for j in range(D // 16):
                    s = pl.ds(j * 16, 16)
                    a_v[p, i, s] = (a_v[p, i, s] + b_v[p, i, s]) * 0.5
                return carry

            lax.fori_loop(0, PC, row, 0)

        def consume(k, p):
            for d in gather_descs(k, p):
                d.wait()
            compute(p)
            pltpu.async_copy(av[p],
                             out_hbm.at[pl.ds(N_NODES + base + k * PC, PC), :],
                             ssem[p])

        def wait_store(k, p):
            pltpu.make_async_copy(av[p],
                                  out_hbm.at[pl.ds(N_NODES + base + k * PC, PC), :],
                                  ssem[p]).wait()

        fire(0, 0)

        def pipe(t, carry):
            k1 = 2 * t + 1

            @pl.when(t >= 1)
            def _():
                wait_store(k1 - 2, 1)

            fire(k1, 1)
            consume(k1 - 1, 0)

            k2 = 2 * t + 2
            wait_store(k2 - 2, 0)
            fire(k2, 0)
            consume(k2 - 1, 1)
            return carry

        lax.fori_loop(0, (CPW - 1) // 2, pipe, 0)
        consume(CPW - 1, 0)
        wait_store(CPW - 2, 1)
        wait_store(CPW - 1, 0)

        @pl.when(wp < NTAIL)
        def _():
            tbase = NPW * WSPAN + wp * PT
            av0 = a_v.at[0, pl.ds(0, PT), :]
            bv0 = b_v.at[0, pl.ds(0, PT), :]
            pltpu.sync_copy(i0_hbm.at[pl.ds(tbase, PT)], i0v.at[pl.ds(0, PT)])
            pltpu.sync_copy(i1_hbm.at[pl.ds(tbase, PT)], i1v.at[pl.ds(0, PT)])
            pltpu.async_copy(x_hbm.at[i0v.at[pl.ds(0, PT)]], av0, gsem[0])
            pltpu.async_copy(x_hbm.at[i1v.at[pl.ds(0, PT)]], bv0, gsem[0])
            pltpu.make_async_copy(x_hbm.at[i0v.at[pl.ds(0, PT)]], av0,
                                  gsem[0]).wait()
            pltpu.make_async_copy(x_hbm.at[i1v.at[pl.ds(0, PT)]], bv0,
                                  gsem[0]).wait()

            def trow(i, carry):
                for j in range(D // 16):
                    s = pl.ds(j * 16, 16)
                    a_v[0, i, s] = (a_v[0, i, s] + b_v[0, i, s]) * 0.5
                return carry

            lax.fori_loop(0, PT, trow, 0)
            pltpu.sync_copy(av0, out_hbm.at[pl.ds(N_NODES + tbase, PT), :])


def kernel(X, pool_idx):
    idx0 = pool_idx[:, 0]
    idx1 = pool_idx[:, 1]
    mesh = plsc.VectorSubcoreMesh(core_axis_name="c", subcore_axis_name="s")
    f = pl.kernel(
        _sc_body,
        out_type=jax.ShapeDtypeStruct((N_NODES + N_POOL, D), jnp.float32),
        mesh=mesh,
        scratch_types=[
            pltpu.VMEM((WSPAN,), jnp.int32),
            pltpu.VMEM((WSPAN,), jnp.int32),
            pltpu.VMEM((2, PC, D), jnp.float32),
            pltpu.VMEM((2, PC, D), jnp.float32),
            pltpu.VMEM((2, XC, D), jnp.float32),
            pltpu.SemaphoreType.DMA,
            pltpu.SemaphoreType.DMA,
            pltpu.SemaphoreType.DMA,
            pltpu.SemaphoreType.DMA,
        ],
    )
    return f(X, idx0, idx1)
